# baseline (speedup 1.0000x reference)
import functools

import jax
import jax.numpy as jnp
from jax import lax
from jax.experimental import pallas as pl
from jax.experimental.pallas import tpu as pltpu


def _body(scal_ref, h_ref, w_ref, emb_ref, ls_ref, ne_ref, logits_ref, mask_ref):
    ds = scal_ref[0, 0]
    ts = scal_ref[0, 1]
    h2 = jnp.dot(h_ref[:], w_ref[:], preferred_element_type=jnp.float32)
    bl = jnp.sum(emb_ref[:] * h2[:, None, :], axis=2)
    dist = jnp.exp(ds * (ls_ref[:].astype(jnp.float32) - ts))
    logits = bl + dist
    bb, m = logits.shape
    iota = lax.broadcasted_iota(jnp.int32, (bb, m), 1)
    mask = iota < ne_ref[:]
    logits_ref[:] = jnp.where(mask, logits, jnp.float32(1e-34))
    mask_ref[:] = mask.astype(jnp.int32)


@functools.partial(jax.jit, static_argnames=("interpret",))
def _run(hidden, embeddings, W_proj, distance_scalar, last_seen,
         num_embeddings, timestep, interpret=False):
    B, M, D = embeddings.shape
    Bb = 16
    scal = jnp.stack([distance_scalar.astype(jnp.float32),
                      jnp.asarray(timestep, jnp.float32)]).reshape(1, 2)
    ne2 = num_embeddings.astype(jnp.int32).reshape(B, 1)
    grid = (B // Bb,)
    logits, mask_i = pl.pallas_call(
        _body,
        grid=grid,
        in_specs=[
            pl.BlockSpec((1, 2), lambda i: (0, 0)),
            pl.BlockSpec((Bb, D), lambda i: (i, 0)),
            pl.BlockSpec((D, D), lambda i: (0, 0)),
            pl.BlockSpec((Bb, M, D), lambda i: (i, 0, 0)),
            pl.BlockSpec((Bb, M), lambda i: (i, 0)),
            pl.BlockSpec((Bb, 1), lambda i: (i, 0)),
        ],
        out_specs=[
            pl.BlockSpec((Bb, M), lambda i: (i, 0)),
            pl.BlockSpec((Bb, M), lambda i: (i, 0)),
        ],
        out_shape=[
            jax.ShapeDtypeStruct((B, M), jnp.float32),
            jax.ShapeDtypeStruct((B, M), jnp.int32),
        ],
        compiler_params=pltpu.CompilerParams(
            dimension_semantics=("parallel",)),
        interpret=interpret,
    )(scal, hidden, W_proj, embeddings, last_seen.astype(jnp.int32), ne2)
    return logits, mask_i.astype(jnp.bool_)


def kernel(hidden, embeddings, W_proj, distance_scalar, last_seen,
           num_embeddings, timestep):
    return _run(hidden, embeddings, W_proj, distance_scalar, last_seen,
                num_embeddings, timestep)


# speedup vs baseline: 1.6496x; 1.6496x over previous
import functools

import jax
import jax.numpy as jnp
from jax import lax
from jax.experimental import pallas as pl
from jax.experimental.pallas import tpu as pltpu


def _body(scal_ref, h_ref, w_ref, emb_ref, ls_ref, ne_ref, logits_ref, mask_ref):
    ds = scal_ref[0, 0]
    ts = scal_ref[0, 1]
    h2 = jnp.dot(h_ref[:], w_ref[:], preferred_element_type=jnp.float32)
    h2b = h2.astype(jnp.bfloat16)
    rows = []
    for b in range(h_ref.shape[0]):
        e_b = emb_ref[b].astype(jnp.bfloat16)
        rows.append(lax.dot_general(
            h2b[b:b + 1, :], e_b,
            dimension_numbers=(((1,), (1,)), ((), ())),
            preferred_element_type=jnp.float32))
    bl = jnp.concatenate(rows, axis=0)
    dist = jnp.exp(ds * (ls_ref[:].astype(jnp.float32) - ts))
    logits = bl + dist
    bb, m = logits.shape
    iota = lax.broadcasted_iota(jnp.int32, (bb, m), 1)
    mask = iota < ne_ref[:]
    logits_ref[:] = jnp.where(mask, logits, jnp.float32(1e-34))
    mask_ref[:] = mask.astype(jnp.int32)


@functools.partial(jax.jit, static_argnames=("interpret",))
def _run(hidden, embeddings, W_proj, distance_scalar, last_seen,
         num_embeddings, timestep, interpret=False):
    B, M, D = embeddings.shape
    Bb = 16
    scal = jnp.stack([distance_scalar.astype(jnp.float32),
                      jnp.asarray(timestep, jnp.float32)]).reshape(1, 2)
    ne2 = num_embeddings.astype(jnp.int32).reshape(B, 1)
    grid = (B // Bb,)
    logits, mask_i = pl.pallas_call(
        _body,
        grid=grid,
        in_specs=[
            pl.BlockSpec((1, 2), lambda i: (0, 0)),
            pl.BlockSpec((Bb, D), lambda i: (i, 0)),
            pl.BlockSpec((D, D), lambda i: (0, 0)),
            pl.BlockSpec((Bb, M, D), lambda i: (i, 0, 0)),
            pl.BlockSpec((Bb, M), lambda i: (i, 0)),
            pl.BlockSpec((Bb, 1), lambda i: (i, 0)),
        ],
        out_specs=[
            pl.BlockSpec((Bb, M), lambda i: (i, 0)),
            pl.BlockSpec((Bb, M), lambda i: (i, 0)),
        ],
        out_shape=[
            jax.ShapeDtypeStruct((B, M), jnp.float32),
            jax.ShapeDtypeStruct((B, M), jnp.int32),
        ],
        compiler_params=pltpu.CompilerParams(
            dimension_semantics=("parallel",)),
        interpret=interpret,
    )(scal, hidden, W_proj, embeddings, last_seen.astype(jnp.int32), ne2)
    return logits, mask_i.astype(jnp.bool_)


def kernel(hidden, embeddings, W_proj, distance_scalar, last_seen,
           num_embeddings, timestep):
    return _run(hidden, embeddings, W_proj, distance_scalar, last_seen,
                num_embeddings, timestep)


# speedup vs baseline: 2.2089x; 1.3391x over previous
import functools

import jax
import jax.numpy as jnp
from jax import lax
from jax.experimental import pallas as pl
from jax.experimental.pallas import tpu as pltpu


def _body(scal_ref, h_ref, w_ref, emb_ref, ls_ref, ne_ref, logits_ref, mask_ref):
    ds = scal_ref[0, 0]
    ts = scal_ref[0, 1]
    h2 = jnp.dot(h_ref[:], w_ref[:], preferred_element_type=jnp.float32)
    h2b = h2.astype(jnp.bfloat16)
    rows = []
    for b in range(h_ref.shape[0]):
        e_b = emb_ref[b].astype(jnp.bfloat16)
        rows.append(lax.dot_general(
            h2b[b:b + 1, :], e_b,
            dimension_numbers=(((1,), (1,)), ((), ())),
            preferred_element_type=jnp.float32))
    bl = jnp.concatenate(rows, axis=0)
    dist = jnp.exp(ds * (ls_ref[:].astype(jnp.float32) - ts))
    logits = bl + dist
    bb, m = logits.shape
    iota = lax.broadcasted_iota(jnp.int32, (bb, m), 1)
    mask = iota < ne_ref[:]
    logits_ref[:] = jnp.where(mask, logits, jnp.float32(1e-34))
    mask_ref[:] = mask.astype(jnp.int32)


@functools.partial(jax.jit, static_argnames=("interpret",))
def _run(hidden, embeddings, W_proj, distance_scalar, last_seen,
         num_embeddings, timestep, interpret=False):
    B, M, D = embeddings.shape
    Bb = 32
    scal = jnp.stack([distance_scalar.astype(jnp.float32),
                      jnp.asarray(timestep, jnp.float32)]).reshape(1, 2)
    ne2 = num_embeddings.astype(jnp.int32).reshape(B, 1)
    grid = (B // Bb,)
    logits, mask_i = pl.pallas_call(
        _body,
        grid=grid,
        in_specs=[
            pl.BlockSpec((1, 2), lambda i: (0, 0)),
            pl.BlockSpec((Bb, D), lambda i: (i, 0)),
            pl.BlockSpec((D, D), lambda i: (0, 0)),
            pl.BlockSpec((Bb, M, D), lambda i: (i, 0, 0)),
            pl.BlockSpec((Bb, M), lambda i: (i, 0)),
            pl.BlockSpec((Bb, 1), lambda i: (i, 0)),
        ],
        out_specs=[
            pl.BlockSpec((Bb, M), lambda i: (i, 0)),
            pl.BlockSpec((Bb, M), lambda i: (i, 0)),
        ],
        out_shape=[
            jax.ShapeDtypeStruct((B, M), jnp.float32),
            jax.ShapeDtypeStruct((B, M), jnp.int32),
        ],
        compiler_params=pltpu.CompilerParams(
            dimension_semantics=("parallel",)),
        interpret=interpret,
    )(scal, hidden, W_proj, embeddings, last_seen.astype(jnp.int32), ne2)
    return logits, mask_i.astype(jnp.bool_)


def kernel(hidden, embeddings, W_proj, distance_scalar, last_seen,
           num_embeddings, timestep):
    return _run(hidden, embeddings, W_proj, distance_scalar, last_seen,
                num_embeddings, timestep)


# speedup vs baseline: 2.5386x; 1.1493x over previous
import functools

import jax
import jax.numpy as jnp
from jax import lax
from jax.experimental import pallas as pl
from jax.experimental.pallas import tpu as pltpu


def _body(scal_ref, h_ref, w_ref, emb_ref, ls_ref, ne_ref, logits_ref, mask_ref):
    ds = scal_ref[0, 0]
    ts = scal_ref[0, 1]
    h2 = jnp.dot(h_ref[:], w_ref[:], preferred_element_type=jnp.float32)
    h2b = h2.astype(jnp.bfloat16)
    rows = []
    for b in range(h_ref.shape[0]):
        e_b = emb_ref[b].astype(jnp.bfloat16)
        rows.append(lax.dot_general(
            h2b[b:b + 1, :], e_b,
            dimension_numbers=(((1,), (1,)), ((), ())),
            preferred_element_type=jnp.float32))
    bl = jnp.concatenate(rows, axis=0)
    dist = jnp.exp(ds * (ls_ref[:].astype(jnp.float32) - ts))
    logits = bl + dist
    bb, m = logits.shape
    iota = lax.broadcasted_iota(jnp.int32, (bb, m), 1)
    mask = iota < ne_ref[:]
    logits_ref[:] = jnp.where(mask, logits, jnp.float32(1e-34))
    mask_ref[:] = mask.astype(jnp.int32)


@functools.partial(jax.jit, static_argnames=("interpret",))
def _run(hidden, embeddings, W_proj, distance_scalar, last_seen,
         num_embeddings, timestep, interpret=False):
    B, M, D = embeddings.shape
    Bb = 64
    scal = jnp.stack([distance_scalar.astype(jnp.float32),
                      jnp.asarray(timestep, jnp.float32)]).reshape(1, 2)
    ne2 = num_embeddings.astype(jnp.int32).reshape(B, 1)
    grid = (B // Bb,)
    logits, mask_i = pl.pallas_call(
        _body,
        grid=grid,
        in_specs=[
            pl.BlockSpec((1, 2), lambda i: (0, 0)),
            pl.BlockSpec((Bb, D), lambda i: (i, 0)),
            pl.BlockSpec((D, D), lambda i: (0, 0)),
            pl.BlockSpec((Bb, M, D), lambda i: (i, 0, 0)),
            pl.BlockSpec((Bb, M), lambda i: (i, 0)),
            pl.BlockSpec((Bb, 1), lambda i: (i, 0)),
        ],
        out_specs=[
            pl.BlockSpec((Bb, M), lambda i: (i, 0)),
            pl.BlockSpec((Bb, M), lambda i: (i, 0)),
        ],
        out_shape=[
            jax.ShapeDtypeStruct((B, M), jnp.float32),
            jax.ShapeDtypeStruct((B, M), jnp.int32),
        ],
        compiler_params=pltpu.CompilerParams(
            dimension_semantics=("parallel",),
            vmem_limit_bytes=100 * 1024 * 1024),
        interpret=interpret,
    )(scal, hidden, W_proj, embeddings, last_seen.astype(jnp.int32), ne2)
    return logits, mask_i.astype(jnp.bool_)


def kernel(hidden, embeddings, W_proj, distance_scalar, last_seen,
           num_embeddings, timestep):
    return _run(hidden, embeddings, W_proj, distance_scalar, last_seen,
                num_embeddings, timestep)
